# baseline (speedup 1.0000x reference)
import functools

import jax
import jax.numpy as jnp
from jax.experimental import pallas as pl
from jax.experimental.pallas import tpu as pltpu

LANE = 128
SUBLANE = 8
TILE_B = 8192
VMEM_LIMIT_BYTES = 100 * 1024 * 1024


def _round_up(x, m):
    return (x + m - 1) // m * m


def _mlp_body(x, w1_ref, b1_ref, w2_ref, b2_ref):
    h = jnp.dot(x, w1_ref[...], preferred_element_type=jnp.float32)
    h = jnp.maximum(h + b1_ref[...], 0.0)
    out = jnp.dot(h, w2_ref[...], preferred_element_type=jnp.float32)
    return out + b2_ref[...]


def _manual_kernel(n_steps, tb,
                   x_hbm, w1_ref, b1_ref, w2_ref, b2_ref, o_hbm,
                   x_buf, o_buf, in_sems, out_sems):
    base = pl.program_id(0) * n_steps

    def in_copy(k, slot):
        return pltpu.make_async_copy(
            x_hbm.at[pl.ds((base + k) * tb, tb), :],
            x_buf.at[slot], in_sems.at[slot])

    def out_copy(k, slot):
        return pltpu.make_async_copy(
            o_buf.at[slot],
            o_hbm.at[pl.ds((base + k) * tb, tb), :], out_sems.at[slot])

    in_copy(0, 0).start()
    for k in range(n_steps):
        slot = k % 2
        if k + 1 < n_steps:
            in_copy(k + 1, 1 - slot).start()
        in_copy(k, slot).wait()
        if k >= 2:
            out_copy(k - 2, slot).wait()
        o_buf[slot] = _mlp_body(x_buf[slot], w1_ref, b1_ref, w2_ref, b2_ref)
        out_copy(k, slot).start()
    for k in range(max(0, n_steps - 2), n_steps):
        out_copy(k, k % 2).wait()


def _auto_kernel(x_ref, w1_ref, b1_ref, w2_ref, b2_ref, o_ref):
    o_ref[...] = _mlp_body(x_ref[...], w1_ref, b1_ref, w2_ref, b2_ref)


def kernel(x, w1, b1, w2p, b2p):
    B, D = x.shape
    H = w1.shape[1]
    A = w2p.shape[1]
    A_pad = max(_round_up(A, LANE), LANE)
    if A_pad != A:
        w2p = jnp.pad(w2p, ((0, 0), (0, A_pad - A)))
        b2p = jnp.pad(b2p, ((0, 0), (0, A_pad - A)))

    vmem = pltpu.MemorySpace.VMEM
    hbm = pltpu.MemorySpace.HBM
    n_tiles = B // TILE_B if B % TILE_B == 0 else 0

    if n_tiles >= 4 and n_tiles % 2 == 0:
        tb = TILE_B
        n_steps = n_tiles // 2
        out = pl.pallas_call(
            functools.partial(_manual_kernel, n_steps, tb),
            out_shape=jax.ShapeDtypeStruct((B, A_pad), jnp.float32),
            grid=(2,),
            in_specs=[
                pl.BlockSpec(memory_space=hbm),
                pl.BlockSpec((D, H), lambda c: (0, 0)),
                pl.BlockSpec((1, H), lambda c: (0, 0)),
                pl.BlockSpec((H, A_pad), lambda c: (0, 0)),
                pl.BlockSpec((1, A_pad), lambda c: (0, 0)),
            ],
            out_specs=pl.BlockSpec(memory_space=hbm),
            scratch_shapes=[
                pltpu.VMEM((2, tb, D), jnp.float32),
                pltpu.VMEM((2, tb, A_pad), jnp.float32),
                pltpu.SemaphoreType.DMA((2,)),
                pltpu.SemaphoreType.DMA((2,)),
            ],
            compiler_params=pltpu.CompilerParams(
                dimension_semantics=("parallel",),
                vmem_limit_bytes=VMEM_LIMIT_BYTES,
            ),
        )(x, w1, b1, w2p, b2p)
        return out[:, :A]

    tb = min(TILE_B, _round_up(B, SUBLANE))
    B_pad = _round_up(B, tb)
    if B_pad != B:
        x = jnp.pad(x, ((0, B_pad - B), (0, 0)))
    n_tiles = B_pad // tb
    out = pl.pallas_call(
        _auto_kernel,
        out_shape=jax.ShapeDtypeStruct((B_pad, A_pad), jnp.float32),
        grid=(n_tiles,),
        in_specs=[
            pl.BlockSpec((tb, D), lambda i: (i, 0)),
            pl.BlockSpec((D, H), lambda i: (0, 0)),
            pl.BlockSpec((1, H), lambda i: (0, 0)),
            pl.BlockSpec((H, A_pad), lambda i: (0, 0)),
            pl.BlockSpec((1, A_pad), lambda i: (0, 0)),
        ],
        out_specs=pl.BlockSpec((tb, A_pad), lambda i: (i, 0)),
        compiler_params=pltpu.CompilerParams(
            dimension_semantics=("parallel",),
            vmem_limit_bytes=VMEM_LIMIT_BYTES,
        ),
    )(x, w1, b1, w2p, b2p)
    return out[:B, :A]


# speedup vs baseline: 1.1127x; 1.1127x over previous
import jax
import jax.numpy as jnp
from jax.experimental import pallas as pl
from jax.experimental.pallas import tpu as pltpu

LANE = 128
SUBLANE = 8
TILE_B = 8192
VMEM_LIMIT_BYTES = 100 * 1024 * 1024


def _round_up(x, m):
    return (x + m - 1) // m * m


M_CHUNK = 2048


def _mlp_kernel(x_ref, w1_ref, b1_ref, w2_ref, b2_ref, o_ref):
    w1 = w1_ref[...].astype(jnp.bfloat16)
    w2 = w2_ref[...].astype(jnp.bfloat16)
    b1b = b1_ref[...].astype(jnp.bfloat16)
    b2 = b2_ref[...]
    tb = x_ref.shape[0]
    for m0 in range(0, tb, M_CHUNK):
        mc = min(M_CHUNK, tb - m0)
        x = x_ref[pl.ds(m0, mc), :].astype(jnp.bfloat16)
        h = jnp.dot(x, w1, preferred_element_type=jnp.float32)
        h = jnp.maximum(h.astype(jnp.bfloat16) + b1b, jnp.bfloat16(0.0))
        out = jnp.dot(h, w2, preferred_element_type=jnp.float32)
        o_ref[pl.ds(m0, mc), :] = out + b2


def kernel(x, w1, b1, w2p, b2p):
    B, D = x.shape
    H = w1.shape[1]
    A = w2p.shape[1]
    A_pad = max(_round_up(A, LANE), LANE)
    if A_pad != A:
        w2p = jnp.pad(w2p, ((0, 0), (0, A_pad - A)))
        b2p = jnp.pad(b2p, ((0, 0), (0, A_pad - A)))

    tb = min(TILE_B, _round_up(B, SUBLANE))
    B_pad = _round_up(B, tb)
    if B_pad != B:
        x = jnp.pad(x, ((0, B_pad - B), (0, 0)))
    n_tiles = B_pad // tb

    out = pl.pallas_call(
        _mlp_kernel,
        out_shape=jax.ShapeDtypeStruct((B_pad, A_pad), jnp.float32),
        grid=(n_tiles,),
        in_specs=[
            pl.BlockSpec((tb, D), lambda i: (i, 0)),
            pl.BlockSpec((D, H), lambda i: (0, 0)),
            pl.BlockSpec((1, H), lambda i: (0, 0)),
            pl.BlockSpec((H, A_pad), lambda i: (0, 0)),
            pl.BlockSpec((1, A_pad), lambda i: (0, 0)),
        ],
        out_specs=pl.BlockSpec((tb, A_pad), lambda i: (i, 0)),
        compiler_params=pltpu.CompilerParams(
            dimension_semantics=("parallel",),
            vmem_limit_bytes=VMEM_LIMIT_BYTES,
        ),
    )(x, w1, b1, w2p, b2p)

    return out[:B, :A]
